# Initial kernel scaffold; baseline (speedup 1.0000x reference)
import jax
import jax.numpy as jnp
from jax.experimental import pallas as pl


def kernel(input, W_qkv, b_qkv, W_down, b_down, W_up, b_up):
    raise NotImplementedError("write your pallas kernel here")


# speedup vs baseline: 3.2335x; 3.2335x over previous
import jax
import jax.numpy as jnp
from jax.experimental import pallas as pl
from jax.experimental.pallas import tpu as pltpu

_S = 12
_P = 12
_D3 = 48
_BM = 3456
_HALO = 8


def _block_kernel(M, x_ref, fh_ref, bh_ref, x01_ref, wq_ref, wk_ref, wv_ref,
                  bq_ref, bk_ref, bv_ref, wd_ref, bd_ref, wu_ref, bu_ref,
                  o_ref):
    i = pl.program_id(1)
    BME = _BM + 2 * _HALO

    def dot(a, b):
        return jax.lax.dot_general(a, b, (((1,), (0,)), ((), ())),
                                   preferred_element_type=jnp.float32)

    xc = x_ref[0]
    fh = fh_ref[0, 0]
    bh = bh_ref[0, 0]
    x8 = x01_ref[0]

    wq = wq_ref[...]
    wv = wv_ref[...]

    q_ext = jnp.concatenate([dot(fh, wq), dot(xc, wq), dot(bh, wq)],
                            axis=0) + bq_ref[...]
    v_ext = jnp.concatenate([dot(fh, wv), dot(xc, wv), dot(bh, wv)],
                            axis=0) + bv_ref[...]

    rows = jax.lax.broadcasted_iota(jnp.int32, (BME, _D3), 0)
    gi = i * _BM - _HALO + rows
    valid = (gi >= 0) & (gi < M)
    v_m = jnp.where(valid, v_ext, 0.0)

    k8 = dot(x8, wk_ref[...]) + bk_ref[...]
    v8 = dot(x8, wv) + bv_ref[...]
    kd = k8[0:1, :] - k8[1:2, :]
    kd2 = jnp.concatenate([kd, -kd], axis=0)
    s2 = jax.lax.dot_general(q_ext, kd2, (((1,), (1,)), ((), ())),
                             preferred_element_type=jnp.float32)
    p2 = jax.nn.sigmoid(s2)
    att = dot(p2, v8[0:2, :])

    yd0 = dot(v_m, wd_ref[0])
    yd1 = dot(v_m, wd_ref[1])
    yd2 = dot(v_m, wd_ref[2])
    mid_c = att + bd_ref[...] + yd1
    midv = mid_c[1:BME - 1] + yd0[0:BME - 2] + yd2[2:BME]
    midv = jnp.where(valid[1:BME - 1], midv, 0.0)

    yu0 = dot(midv, wu_ref[0])
    yu1 = dot(midv, wu_ref[1])
    yu2 = dot(midv, wu_ref[2])
    out = (yu0[_HALO - 2:_HALO - 2 + _BM] + yu1[_HALO - 1:_HALO - 1 + _BM]
           + yu2[_HALO:_HALO + _BM] + bu_ref[...])
    o_ref[0] = out


def kernel(input, W_qkv, b_qkv, W_down, b_down, W_up, b_up):
    B, C, H, W = input.shape
    xu = input.reshape(B, C, _S, _P, _S, _P)
    xu = jnp.transpose(xu, (0, 1, 3, 5, 2, 4)).reshape(B, C * _P * _P, _S * _S)
    x = xu.reshape(B, _S * _S, -1, _P * _P)
    x = jnp.transpose(x, (0, 2, 1, 3)).reshape(B, -1, _S * _S)
    M = x.shape[1]
    nb = M // _BM

    xr = x.reshape(B, nb, _BM, _S * _S)
    z8 = jnp.zeros((B, 1, _HALO, _S * _S), x.dtype)
    fh = jnp.concatenate([z8, xr[:, :-1, _BM - _HALO:, :]], axis=1)
    bh = jnp.concatenate([xr[:, 1:, :_HALO, :], z8], axis=1)
    x01 = x[:, :_HALO, :]

    WT = W_qkv.T
    wq, wk, wv = WT[:, :_D3], WT[:, _D3:2 * _D3], WT[:, 2 * _D3:]
    bq = b_qkv[:_D3].reshape(1, _D3)
    bk = b_qkv[_D3:2 * _D3].reshape(1, _D3)
    bv = b_qkv[2 * _D3:].reshape(1, _D3)
    wd = jnp.transpose(W_down, (2, 1, 0))
    wu = jnp.transpose(W_up, (2, 1, 0))
    bd = b_down.reshape(1, _D3)
    bu = b_up.reshape(1, _D3)

    def full(shp, nd):
        return pl.BlockSpec(shp, (lambda b, i: (0,) * nd))

    out = pl.pallas_call(
        lambda *refs: _block_kernel(M, *refs),
        grid=(B, nb),
        in_specs=[
            pl.BlockSpec((1, _BM, _S * _S), lambda b, i: (b, i, 0)),
            pl.BlockSpec((1, 1, _HALO, _S * _S), lambda b, i: (b, i, 0, 0)),
            pl.BlockSpec((1, 1, _HALO, _S * _S), lambda b, i: (b, i, 0, 0)),
            pl.BlockSpec((1, _HALO, _S * _S), lambda b, i: (b, 0, 0)),
            full((_S * _S, _D3), 2),
            full((_S * _S, _D3), 2),
            full((_S * _S, _D3), 2),
            full((1, _D3), 2),
            full((1, _D3), 2),
            full((1, _D3), 2),
            full((3, _D3, _D3), 3),
            full((1, _D3), 2),
            full((3, _D3, _D3), 3),
            full((1, _D3), 2),
        ],
        out_specs=pl.BlockSpec((1, _BM, _D3), lambda b, i: (b, i, 0)),
        out_shape=jax.ShapeDtypeStruct((B, M, _D3), jnp.float32),
        compiler_params=pltpu.CompilerParams(
            dimension_semantics=("arbitrary", "arbitrary")),
    )(x, fh, bh, x01, wq, wk, wv, bq, bk, bv, wd, bd, wu, bu)

    out = out.reshape(B, -1, _S * _S, _P * _P)
    out = jnp.transpose(out, (0, 2, 1, 3))
    return out.reshape(B, -1, _S * _P, _S * _P)
